# baseline (speedup 1.0000x reference)
import functools

import jax
import jax.numpy as jnp
from jax.experimental import pallas as pl
from jax.experimental.pallas import tpu as pltpu


def _dense_fused_kernel(x_ref, s1_ref, b1_ref, w1_ref, b2_ref, w2r_ref,
                        o_ref, *, H, W, Cin, C1, Cout):
    HW = H * W
    x2 = x_ref[0]
    o_ref[0, :Cin] = x2

    h = jnp.maximum(x2 * s1_ref[...] + b1_ref[...], 0.0)

    t = jax.lax.dot_general(w1_ref[...], h, (((0,), (0,)), ((), ())),
                            preferred_element_type=jnp.float32)
    t = jnp.maximum(t + b2_ref[...], 0.0)

    col = jax.lax.broadcasted_iota(jnp.int32, (C1, HW), 1) % W
    t_l = jnp.where(col >= 1, pltpu.roll(t, 1, axis=1), 0.0)
    t_r = jnp.where(col <= W - 2, pltpu.roll(t, HW - 1, axis=1), 0.0)
    cat = jnp.concatenate([t_l, t, t_r], axis=0)

    p = jax.lax.dot_general(w2r_ref[...], cat, (((1,), (0,)), ((), ())),
                            preferred_element_type=jnp.float32)

    lane = jax.lax.broadcasted_iota(jnp.int32, (Cout, HW), 1)
    y = p[Cout:2 * Cout]
    y = y + jnp.where(lane >= W, pltpu.roll(p[:Cout], W, axis=1), 0.0)
    y = y + jnp.where(lane < HW - W,
                      pltpu.roll(p[2 * Cout:], HW - W, axis=1), 0.0)

    o_ref[0, Cin:] = y


def kernel(x, s1, b1, w1_eff, b2, w2):
    N, Cin, H, W = x.shape
    C1 = w1_eff.shape[1]
    Cout = w2.shape[1]
    HW = H * W

    x_flat = x.reshape(N, Cin, HW)
    s1c = s1.reshape(Cin, 1)
    b1c = b1.reshape(Cin, 1)
    b2c = b2.reshape(C1, 1)
    w2r = w2.reshape(3, 3 * C1, Cout).transpose(0, 2, 1).reshape(3 * Cout,
                                                                 3 * C1)

    out_flat = pl.pallas_call(
        functools.partial(_dense_fused_kernel, H=H, W=W, Cin=Cin, C1=C1,
                          Cout=Cout),
        out_shape=jax.ShapeDtypeStruct((N, Cin + Cout, HW), jnp.float32),
        grid=(N,),
        in_specs=[
            pl.BlockSpec((1, Cin, HW), lambda n: (n, 0, 0)),
            pl.BlockSpec((Cin, 1), lambda n: (0, 0)),
            pl.BlockSpec((Cin, 1), lambda n: (0, 0)),
            pl.BlockSpec((Cin, C1), lambda n: (0, 0)),
            pl.BlockSpec((C1, 1), lambda n: (0, 0)),
            pl.BlockSpec((3 * Cout, 3 * C1), lambda n: (0, 0)),
        ],
        out_specs=pl.BlockSpec((1, Cin + Cout, HW), lambda n: (n, 0, 0)),
        compiler_params=pltpu.CompilerParams(
            dimension_semantics=("parallel",)),
    )(x_flat, s1c, b1c, w1_eff, b2c, w2r)

    return out_flat.reshape(N, Cin + Cout, H, W)
